# Initial kernel scaffold; baseline (speedup 1.0000x reference)
import jax
import jax.numpy as jnp
from jax.experimental import pallas as pl


def kernel(pred_logits, pred_boxes, target_sizes):
    raise NotImplementedError("write your pallas kernel here")


# speedup vs baseline: 2.6319x; 2.6319x over previous
import functools

import jax
import jax.numpy as jnp
from jax.experimental import pallas as pl
from jax.experimental.pallas import tpu as pltpu

N_CLASSES_KEPT = 90
K = 300
THRESHOLD = 0.05
IOU_THRESHOLD = 0.85
NEG_KEY = -(2**31)

B = 8
N = 20000
CH = 2000
G = N // CH
CHP = 2048
NP = G * CHP


def _key_from_f32(m):
    bits = jax.lax.bitcast_convert_type(m, jnp.int32)
    return jnp.where(bits >= 0, bits, bits ^ jnp.int32(0x7FFFFFFF))


def _f32_from_key(k):
    bits = jnp.where(k >= 0, k, k ^ jnp.int32(0x7FFFFFFF))
    return jax.lax.bitcast_convert_type(bits, jnp.float32)


def _body_a(x_ref, keys_ref, amax_ref):
    x = x_ref[0][:, :N_CLASSES_KEPT]
    m = jnp.max(x, axis=-1, keepdims=True)
    cls = jax.lax.broadcasted_iota(jnp.int32, x.shape, 1)
    a = jnp.min(jnp.where(x == m, cls, jnp.int32(N_CLASSES_KEPT)), axis=-1)
    key = _key_from_f32(m[:, 0])
    pad = jnp.full((CHP - CH,), jnp.int32(NEG_KEY), dtype=jnp.int32)
    keys_ref[0, 0, 0] = jnp.concatenate([key, pad])
    amax_ref[0, 0, 0] = jnp.concatenate([a.astype(jnp.int32),
                                         jnp.zeros((CHP - CH,), jnp.int32)])


def _stage_a(pred_logits):
    keys3, amax3 = pl.pallas_call(
        _body_a,
        grid=(B, G),
        in_specs=[pl.BlockSpec((1, CH, 91), lambda b, g: (b, g, 0))],
        out_specs=[pl.BlockSpec((1, 1, 1, CHP), lambda b, g: (b, g, 0, 0)),
                   pl.BlockSpec((1, 1, 1, CHP), lambda b, g: (b, g, 0, 0))],
        out_shape=[jax.ShapeDtypeStruct((B, G, 1, CHP), jnp.int32),
                   jax.ShapeDtypeStruct((B, G, 1, CHP), jnp.int32)],
    )(pred_logits)
    return keys3.reshape(B, NP), amax3.reshape(B, NP)


def _body_b(keys_ref, vstar_ref, k1_ref):
    keys = keys_ref[...]
    sgn = jnp.int32(-(2**31))

    def it(t, uv):
        cand = uv | (jnp.int32(1) << (jnp.int32(31) - t))
        scand = cand ^ sgn
        c = jnp.sum((keys >= scand[:, None, None]).astype(jnp.int32), axis=(1, 2))
        return jnp.where(c >= K, cand, uv)

    uv = jax.lax.fori_loop(0, 32, it, jnp.zeros((B,), jnp.int32))
    vstar = uv ^ sgn
    k1 = jnp.sum((keys > vstar[:, None, None]).astype(jnp.int32), axis=(1, 2))
    vstar_ref[0, :] = vstar
    k1_ref[0, :] = k1


def _stage_b(keys3):
    return pl.pallas_call(
        _body_b,
        out_shape=[jax.ShapeDtypeStruct((1, B), jnp.int32),
                   jax.ShapeDtypeStruct((1, B), jnp.int32)],
    )(keys3)


def _stage_c(keys, amax, pred_boxes, vstar, k1):
    gt = keys > vstar[:, None]
    eq = keys == vstar[:, None]
    eq_rank = jnp.cumsum(eq.astype(jnp.int32), axis=1) - 1
    sel = gt | (eq & (eq_rank < (K - k1)[:, None]))

    def compact(selrow):
        return jnp.nonzero(selrow, size=K, fill_value=0)[0].astype(jnp.int32)

    cand_pidx = jax.vmap(compact)(sel)
    cand_key = jnp.take_along_axis(keys, cand_pidx, axis=1)
    cand_lab = jnp.take_along_axis(amax, cand_pidx, axis=1)
    cand_idx = (cand_pidx // CHP) * CH + (cand_pidx % CHP)
    cand_box = jnp.take_along_axis(pred_boxes, cand_idx[..., None], axis=1)
    return cand_idx, cand_key, cand_lab, cand_box


def _body_d1(idx_ref, key_ref, lab_ref, box_ref, ts_ref,
             s_ref, box_out_ref, lab_out_ref, valid_ref, iou_ref):
    key_row = key_ref[0]
    idx_row = idx_ref[0]
    lab_row = lab_ref[0].astype(jnp.float32)
    key_sub = jnp.transpose(key_row)
    idx_sub = jnp.transpose(idx_row)

    gt = (key_row > key_sub) | ((key_row == key_sub) & (idx_row < idx_sub))
    rank_sub = jnp.sum(gt.astype(jnp.int32), axis=1, keepdims=True)
    rank_row = jnp.transpose(rank_sub)
    p_sub = jax.lax.broadcasted_iota(jnp.int32, (K, 1), 0)
    onehot = rank_row == p_sub

    def permute(x_row):
        return jnp.sum(jnp.where(onehot, x_row, 0.0), axis=1, keepdims=True)

    m_sub = permute(_f32_from_key(key_row))
    s_sub = jax.nn.sigmoid(m_sub)
    lab_sub = permute(lab_row)
    bx = box_ref[0]
    cx = permute(jnp.transpose(bx[:, 0:1]))
    cy = permute(jnp.transpose(bx[:, 1:2]))
    w = permute(jnp.transpose(bx[:, 2:3]))
    h = permute(jnp.transpose(bx[:, 3:4]))

    b0 = cx - w * 0.5; b1 = cy - h * 0.5
    b2 = cx + w * 0.5; b3 = cy + h * 0.5
    ts = ts_ref[0].astype(jnp.float32)
    hgt = ts[0:1, 0:1]; wid = ts[0:1, 1:2]
    b0 = jnp.clip(b0 * wid, 0.0, wid)
    b1 = jnp.clip(b1 * hgt, 0.0, hgt)
    b2 = jnp.clip(b2 * wid, 0.0, wid)
    b3 = jnp.clip(b3 * hgt, 0.0, hgt)

    valid = (s_sub > THRESHOLD) & (b2 > b0) & (b3 > b1)
    mc = jnp.max(jnp.maximum(jnp.maximum(b0, b1), jnp.maximum(b2, b3)),
                 keepdims=True) + 1.0
    offs = lab_sub * mc
    x1 = b0 + offs; y1 = b1 + offs; x2 = b2 + offs; y2 = b3 + offs
    areas = (x2 - x1) * (y2 - y1)
    x1r = jnp.transpose(x1); y1r = jnp.transpose(y1)
    x2r = jnp.transpose(x2); y2r = jnp.transpose(y2)
    xx1 = jnp.maximum(x1, x1r); yy1 = jnp.maximum(y1, y1r)
    xx2 = jnp.minimum(x2, x2r); yy2 = jnp.minimum(y2, y2r)
    iw = jnp.maximum(xx2 - xx1, 0.0); ih = jnp.maximum(yy2 - yy1, 0.0)
    inter = iw * ih
    iou = inter / (areas + jnp.transpose(areas) - inter + 1e-9)

    s_ref[0] = s_sub
    lab_out_ref[0] = lab_sub.astype(jnp.int32) + 1
    box_out_ref[0] = jnp.concatenate([b0, b1, b2, b3], axis=1)
    valid_ref[0] = jnp.transpose(valid)
    iou_ref[...] = iou.reshape(K, 1, 1, K)


def _stage_d1(cand_idx, cand_key, cand_lab, cand_box, target_sizes):
    row3 = lambda b: (b, 0, 0)
    return pl.pallas_call(
        _body_d1,
        grid=(B,),
        in_specs=[pl.BlockSpec((1, 1, K), row3),
                  pl.BlockSpec((1, 1, K), row3),
                  pl.BlockSpec((1, 1, K), row3),
                  pl.BlockSpec((1, K, 4), row3),
                  pl.BlockSpec((1, 1, 2), row3)],
        out_specs=[pl.BlockSpec((1, K, 1), row3),
                   pl.BlockSpec((1, K, 4), row3),
                   pl.BlockSpec((1, K, 1), row3),
                   pl.BlockSpec((1, 1, K), row3),
                   pl.BlockSpec((K, 1, 1, K), lambda b: (0, b, 0, 0))],
        out_shape=[jax.ShapeDtypeStruct((B, K, 1), jnp.float32),
                   jax.ShapeDtypeStruct((B, K, 4), jnp.float32),
                   jax.ShapeDtypeStruct((B, K, 1), jnp.int32),
                   jax.ShapeDtypeStruct((B, 1, K), jnp.bool_),
                   jax.ShapeDtypeStruct((K, B, 1, K), jnp.float32)],
    )(cand_idx.reshape(B, 1, K), cand_key.reshape(B, 1, K),
      cand_lab.reshape(B, 1, K), cand_box, target_sizes.reshape(B, 1, 2))


def _body_d2(valid_ref, iou_ref, keep_ref):
    valid = valid_ref[:, 0, :]
    lane = jax.lax.broadcasted_iota(jnp.int32, (1, K), 1)

    def nms_it(i, suppressed):
        sup_i = jnp.sum(jnp.where(lane == i, suppressed, 0),
                        axis=1, keepdims=True)
        val_i = jnp.sum(jnp.where(lane == i, valid.astype(jnp.int32), 0),
                        axis=1, keepdims=True)
        act = (val_i > 0) & (sup_i == 0)
        row = iou_ref[pl.ds(i, 1)][0, :, 0, :]
        hit = act & (row > IOU_THRESHOLD) & (lane > i)
        return suppressed | hit.astype(jnp.int32)

    suppressed = jax.lax.fori_loop(
        0, K, nms_it, jnp.zeros((B, K), dtype=jnp.int32), unroll=4)
    keep_ref[...] = valid & (suppressed == 0)


def _stage_d2(valid, iou):
    return pl.pallas_call(
        _body_d2,
        out_shape=jax.ShapeDtypeStruct((B, K), jnp.bool_),
    )(valid, iou)


def kernel(pred_logits, pred_boxes, target_sizes):
    keys, amax = _stage_a(pred_logits)
    keys3 = keys.reshape(B, G, CHP)
    vstar, k1 = _stage_b(keys3)
    cand_idx, cand_key, cand_lab, cand_box = _stage_c(
        keys, amax, pred_boxes, vstar[0], k1[0])
    s3, boxes, lab3, valid, iou = _stage_d1(
        cand_idx, cand_key, cand_lab, cand_box, target_sizes)
    keep = _stage_d2(valid, iou)
    return s3.reshape(B, K), boxes, lab3.reshape(B, K), keep
